# baseline (speedup 1.0000x reference)
import functools

import jax
import jax.numpy as jnp
from jax import lax
from jax.experimental import pallas as pl
from jax.experimental.pallas import tpu as pltpu
from jax.experimental.pallas import tpu_sc as plsc

_NC = 2
_NS = 16
_NW = _NC * _NS
_L = 16
_BBLK = 128
_TCHUNK = 40


def kernel(x, anchors):
    b, t = x.shape
    num_anchors = anchors.shape[0]
    xt = x.T.astype(jnp.int32)
    tab_flat = anchors.reshape(num_anchors * 4)

    kblk = b // (2 * _BBLK)
    n_chunks = t // _TCHUNK
    groups = _BBLK // _L

    mesh = plsc.VectorSubcoreMesh(core_axis_name="c", subcore_axis_name="s")

    @functools.partial(
        pl.kernel,
        out_type=jax.ShapeDtypeStruct((t, kblk, 8, 128), jnp.float32),
        mesh=mesh,
        compiler_params=pltpu.CompilerParams(needs_layout_passes=False),
        scratch_types=[
            pltpu.VMEM((num_anchors * 4,), jnp.float32),
            pltpu.VMEM((2, _TCHUNK, _BBLK), jnp.int32),
            pltpu.VMEM((2, _TCHUNK, 4, 128), jnp.float32),
            pltpu.SemaphoreType.DMA,
            pltpu.SemaphoreType.DMA,
            pltpu.SemaphoreType.DMA,
            pltpu.SemaphoreType.DMA,
        ],
    )
    def _gather(xt_hbm, tab_hbm, out_hbm, tab_v, idx_v, out_v, is0, is1, os0, os1):
        wid = lax.axis_index("s") * _NC + lax.axis_index("c")
        kk = wid // 2
        r0 = (wid % 2) * 4
        isems = (is0, is1)
        osems = (os0, os1)

        def start_in(ch):
            return pltpu.async_copy(
                xt_hbm.at[
                    pl.ds(ch * _TCHUNK, _TCHUNK), pl.ds(wid * _BBLK, _BBLK)
                ],
                idx_v.at[ch % 2],
                isems[ch % 2],
            )

        in_cp = [None] * n_chunks
        out_cp = [None] * n_chunks
        in_cp[0] = start_in(0)
        pltpu.sync_copy(tab_hbm, tab_v)
        for ch in range(n_chunks):
            s = ch % 2
            if ch + 1 < n_chunks:
                in_cp[ch + 1] = start_in(ch + 1)
            in_cp[ch].wait()
            if ch >= 2:
                out_cp[ch - 2].wait()

            @plsc.parallel_loop(0, _TCHUNK, unroll=2)
            def _(tr):
                for g in range(groups):
                    iv4 = idx_v[s, tr, pl.ds(g * _L, _L)] * 4
                    for c in range(4):
                        out_v[s, tr, c, pl.ds(g * _L, _L)] = plsc.load_gather(
                            tab_v, [iv4 + c]
                        )

            out_cp[ch] = pltpu.async_copy(
                out_v.at[s],
                out_hbm.at[pl.ds(ch * _TCHUNK, _TCHUNK), kk, pl.ds(r0, 4)],
                osems[s],
            )
        out_cp[n_chunks - 2].wait()
        out_cp[n_chunks - 1].wait()

    out4d = _gather(xt, tab_flat)
    out = (
        out4d.reshape(t, kblk, 2, 4, 128)
        .transpose(1, 2, 4, 0, 3)
        .reshape(b, t, 4)
    )
    return out


# speedup vs baseline: 1.2222x; 1.2222x over previous
import functools

import jax
import jax.numpy as jnp
from jax import lax
from jax.experimental import pallas as pl
from jax.experimental.pallas import tpu as pltpu
from jax.experimental.pallas import tpu_sc as plsc

_NC = 2
_NS = 16
_NW = _NC * _NS
_L = 16
_BBLK = 128
_TCHUNK = 40


def kernel(x, anchors):
    b, t = x.shape
    num_anchors = anchors.shape[0]
    xt = x.T.astype(jnp.int32)
    tab_flat = anchors.reshape(num_anchors * 4)

    kblk = b // (2 * _BBLK)
    n_chunks = t // _TCHUNK
    groups = _BBLK // _L

    mesh = plsc.VectorSubcoreMesh(core_axis_name="c", subcore_axis_name="s")

    @functools.partial(
        pl.kernel,
        out_type=jax.ShapeDtypeStruct((t, kblk, 8, 128), jnp.float32),
        mesh=mesh,
        compiler_params=pltpu.CompilerParams(needs_layout_passes=False),
        scratch_types=[
            pltpu.VMEM((num_anchors * 4,), jnp.float32),
            pltpu.VMEM((2, _TCHUNK, _BBLK), jnp.int32),
            pltpu.VMEM((2, _TCHUNK, 4, 128), jnp.float32),
            pltpu.SemaphoreType.DMA,
            pltpu.SemaphoreType.DMA,
            pltpu.SemaphoreType.DMA,
            pltpu.SemaphoreType.DMA,
        ],
    )
    def _gather(xt_hbm, tab_hbm, out_hbm, tab_v, idx_v, out_v, is0, is1, os0, os1):
        wid = lax.axis_index("s") * _NC + lax.axis_index("c")
        kk = wid // 2
        r0 = (wid % 2) * 4
        isems = (is0, is1)
        osems = (os0, os1)

        def start_in(ch):
            return pltpu.async_copy(
                xt_hbm.at[
                    pl.ds(ch * _TCHUNK, _TCHUNK), pl.ds(wid * _BBLK, _BBLK)
                ],
                idx_v.at[ch % 2],
                isems[ch % 2],
            )

        in_cp = [None] * n_chunks
        out_cp = [None] * n_chunks
        in_cp[0] = start_in(0)
        pltpu.sync_copy(tab_hbm, tab_v)
        for ch in range(n_chunks):
            s = ch % 2
            if ch + 1 < n_chunks:
                in_cp[ch + 1] = start_in(ch + 1)
            in_cp[ch].wait()
            if ch >= 2:
                out_cp[ch - 2].wait()

            @plsc.parallel_loop(0, _TCHUNK * groups, unroll=4)
            def _(i):
                tr = i // groups
                g = i % groups
                iv4 = idx_v[s, tr, pl.ds(g * _L, _L)] * 4
                for c in range(4):
                    out_v[s, tr, c, pl.ds(g * _L, _L)] = plsc.load_gather(
                        tab_v, [iv4 + c]
                    )

            out_cp[ch] = pltpu.async_copy(
                out_v.at[s],
                out_hbm.at[pl.ds(ch * _TCHUNK, _TCHUNK), kk, pl.ds(r0, 4)],
                osems[s],
            )
        out_cp[n_chunks - 2].wait()
        out_cp[n_chunks - 1].wait()

    out4d = _gather(xt, tab_flat)
    out = (
        out4d.reshape(t, kblk, 2, 4, 128)
        .transpose(1, 2, 4, 0, 3)
        .reshape(b, t, 4)
    )
    return out


# speedup vs baseline: 1.2383x; 1.0132x over previous
import functools

import jax
import jax.numpy as jnp
from jax import lax
from jax.experimental import pallas as pl
from jax.experimental.pallas import tpu as pltpu
from jax.experimental.pallas import tpu_sc as plsc

_NC = 2
_NS = 16
_NW = _NC * _NS
_L = 16
_BBLK = 128
_TCHUNK = 40


def kernel(x, anchors):
    b, t = x.shape
    num_anchors = anchors.shape[0]
    xt = x.T.astype(jnp.int32)
    tab_flat = anchors.reshape(num_anchors * 4)

    kblk = b // (2 * _BBLK)
    n_chunks = t // _TCHUNK
    groups = _BBLK // _L

    mesh = plsc.VectorSubcoreMesh(core_axis_name="c", subcore_axis_name="s")

    @functools.partial(
        pl.kernel,
        out_type=jax.ShapeDtypeStruct((t, kblk, 8, 128), jnp.float32),
        mesh=mesh,
        compiler_params=pltpu.CompilerParams(needs_layout_passes=False),
        scratch_types=[
            pltpu.VMEM((num_anchors * 4,), jnp.float32),
            pltpu.VMEM((2, _TCHUNK, _BBLK), jnp.int32),
            pltpu.VMEM((2, _TCHUNK, 4, 128), jnp.float32),
            pltpu.SemaphoreType.DMA,
            pltpu.SemaphoreType.DMA,
            pltpu.SemaphoreType.DMA,
            pltpu.SemaphoreType.DMA,
        ],
    )
    def _gather(xt_hbm, tab_hbm, out_hbm, tab_v, idx_v, out_v, is0, is1, os0, os1):
        wid = lax.axis_index("s") * _NC + lax.axis_index("c")
        kk = wid // 2
        r0 = (wid % 2) * 4
        isems = (is0, is1)
        osems = (os0, os1)

        def start_in(ch):
            return pltpu.async_copy(
                xt_hbm.at[
                    pl.ds(ch * _TCHUNK, _TCHUNK), pl.ds(wid * _BBLK, _BBLK)
                ],
                idx_v.at[ch % 2],
                isems[ch % 2],
            )

        in_cp = [None] * n_chunks
        out_cp = [None] * n_chunks
        in_cp[0] = start_in(0)
        pltpu.sync_copy(tab_hbm, tab_v)
        for ch in range(n_chunks):
            s = ch % 2
            if ch + 1 < n_chunks:
                in_cp[ch + 1] = start_in(ch + 1)
            in_cp[ch].wait()
            if ch >= 2:
                out_cp[ch - 2].wait()

            @plsc.parallel_loop(0, _TCHUNK * groups, unroll=2)
            def _(i):
                tr = i // groups
                g = i % groups
                iv4 = idx_v[s, tr, pl.ds(g * _L, _L)] * 4
                for c in range(4):
                    out_v[s, tr, c, pl.ds(g * _L, _L)] = plsc.load_gather(
                        tab_v, [iv4 + c]
                    )

            out_cp[ch] = pltpu.async_copy(
                out_v.at[s],
                out_hbm.at[pl.ds(ch * _TCHUNK, _TCHUNK), kk, pl.ds(r0, 4)],
                osems[s],
            )
        out_cp[n_chunks - 2].wait()
        out_cp[n_chunks - 1].wait()

    out4d = _gather(xt, tab_flat)
    out = (
        out4d.reshape(t, kblk, 2, 4, 128)
        .transpose(1, 2, 4, 0, 3)
        .reshape(b, t, 4)
    )
    return out
